# baseline (speedup 1.0000x reference)
import math

import jax
import jax.numpy as jnp
from jax.experimental import pallas as pl

ROWS, COLS = 16384, 200
TOTAL = ROWS * COLS
R2, C2 = TOTAL // 128, 128
BLK = 1600
SCALE = 7.0 / math.pi
HALF_PI = math.pi / 2.0


def _body(x_ref, o_ref):
    v = x_ref[...]
    idx = (v * SCALE).astype(jnp.int32)
    o_ref[...] = idx.astype(jnp.float32) * HALF_PI


@jax.jit
def kernel(inputs):
    z = inputs.T.reshape(COLS // 8, 8, ROWS // 128, 128)
    z = z.transpose(0, 2, 1, 3).reshape(R2, C2)
    o = pl.pallas_call(
        _body,
        grid=(R2 // BLK,),
        in_specs=[pl.BlockSpec((BLK, C2), lambda i: (i, 0))],
        out_specs=pl.BlockSpec((BLK, C2), lambda i: (i, 0)),
        out_shape=jax.ShapeDtypeStruct((R2, C2), jnp.float32),
    )(z)
    o = o.reshape(COLS // 8, ROWS // 128, 8, 128).transpose(0, 2, 1, 3)
    return o.reshape(COLS, ROWS).T


# speedup vs baseline: 1.5285x; 1.5285x over previous
import math

import jax
import jax.numpy as jnp
from jax.experimental import pallas as pl

ROWS, COLS = 16384, 200
TOTAL = ROWS * COLS
R2, C2 = TOTAL // 128, 128
BLK = 6400
SCALE = 7.0 / math.pi
HALF_PI = math.pi / 2.0


def _body(x_ref, o_ref):
    v = x_ref[...]
    idx = (v * SCALE).astype(jnp.int32)
    o_ref[...] = idx.astype(jnp.float32) * HALF_PI


@jax.jit
def kernel(inputs):
    z = inputs.T.reshape(COLS // 8, 8, ROWS // 128, 128)
    z = z.transpose(0, 2, 1, 3).reshape(R2, C2)
    o = pl.pallas_call(
        _body,
        grid=(R2 // BLK,),
        in_specs=[pl.BlockSpec((BLK, C2), lambda i: (i, 0))],
        out_specs=pl.BlockSpec((BLK, C2), lambda i: (i, 0)),
        out_shape=jax.ShapeDtypeStruct((R2, C2), jnp.float32),
    )(z)
    o = o.reshape(COLS // 8, ROWS // 128, 8, 128).transpose(0, 2, 1, 3)
    return o.reshape(COLS, ROWS).T


# speedup vs baseline: 1.8335x; 1.1995x over previous
import math

import jax
import jax.numpy as jnp
from jax.experimental import pallas as pl

ROWS, COLS = 16384, 200
TOTAL = ROWS * COLS
R2, C2 = TOTAL // 128, 128
BLK = 12800
SCALE = 7.0 / math.pi
HALF_PI = math.pi / 2.0


def _body(x_ref, o_ref):
    v = x_ref[...]
    idx = (v * SCALE).astype(jnp.int32)
    o_ref[...] = idx.astype(jnp.float32) * HALF_PI


@jax.jit
def kernel(inputs):
    z = inputs.T.reshape(COLS // 8, 8, ROWS // 128, 128)
    z = z.transpose(0, 2, 1, 3).reshape(R2, C2)
    o = pl.pallas_call(
        _body,
        grid=(R2 // BLK,),
        in_specs=[pl.BlockSpec((BLK, C2), lambda i: (i, 0))],
        out_specs=pl.BlockSpec((BLK, C2), lambda i: (i, 0)),
        out_shape=jax.ShapeDtypeStruct((R2, C2), jnp.float32),
    )(z)
    o = o.reshape(COLS // 8, ROWS // 128, 8, 128).transpose(0, 2, 1, 3)
    return o.reshape(COLS, ROWS).T
